# Initial kernel scaffold; baseline (speedup 1.0000x reference)
import jax
import jax.numpy as jnp
from jax import lax
from jax.experimental import pallas as pl
from jax.experimental.pallas import tpu as pltpu
from jax.experimental.pallas import tpu_sc as plsc

NC = 2
NS = 16
NW = NC * NS
CH = 128


def _sc_mesh():
    return plsc.VectorSubcoreMesh(
        core_axis_name="c", subcore_axis_name="s", num_cores=NC, num_subcores=NS
    )


def _deg_kernel(n_pad, nch):
    sr = n_pad // NS

    def body(col_hbm, ones_hbm, zeros_hbm, out_hbm, cidx_v, ones_v, deg_sh, sem):
        c = lax.axis_index("c")
        s = lax.axis_index("s")
        wid = s * NC + c
        pltpu.sync_copy(zeros_hbm, deg_sh.at[pl.ds(s * sr, sr)])
        pltpu.sync_copy(ones_hbm, ones_v)
        pltpu.sync_copy(col_hbm.at[pl.ds(wid * nch, nch)], cidx_v)
        plsc.subcore_barrier()

        def chunk(j, carry):
            pltpu.sync_copy(ones_v, deg_sh.at[cidx_v.at[j]], add=True)
            return carry

        lax.fori_loop(0, nch, chunk, 0)
        plsc.subcore_barrier()
        pltpu.sync_copy(deg_sh.at[pl.ds(s * sr, sr)],
                        out_hbm.at[c, pl.ds(s * sr, sr)])

    return pl.kernel(
        body,
        out_type=jax.ShapeDtypeStruct((NC, n_pad), jnp.float32),
        mesh=_sc_mesh(),
        scratch_types=[
            pltpu.VMEM((nch, CH), jnp.int32),
            pltpu.VMEM((CH,), jnp.float32),
            pltpu.VMEM_SHARED((n_pad,), jnp.float32),
            pltpu.SemaphoreType.DMA,
        ],
    )


def _agg_kernel(n_pad, nch):
    sr = n_pad // NS

    def body(hs_hbm, row_hbm, col_hbm, zeros_hbm, out_hbm,
             ridx_v, cidx_v, rows_v, agg_sh, sem):
        c = lax.axis_index("c")
        s = lax.axis_index("s")
        wid = s * NC + c
        pltpu.sync_copy(zeros_hbm, agg_sh.at[pl.ds(s * sr, sr)])
        pltpu.sync_copy(row_hbm.at[pl.ds(wid * nch, nch)], ridx_v)
        pltpu.sync_copy(col_hbm.at[pl.ds(wid * nch, nch)], cidx_v)
        plsc.subcore_barrier()

        def chunk(j, carry):
            pltpu.async_copy(hs_hbm.at[ridx_v.at[j]], rows_v, sem).wait()
            pltpu.sync_copy(rows_v, agg_sh.at[cidx_v.at[j]], add=True)
            return carry

        lax.fori_loop(0, nch, chunk, 0)
        plsc.subcore_barrier()
        pltpu.sync_copy(agg_sh.at[pl.ds(s * sr, sr)],
                        out_hbm.at[c, pl.ds(s * sr, sr), :])

    return pl.kernel(
        body,
        out_type=jax.ShapeDtypeStruct((NC, n_pad, 128), jnp.float32),
        mesh=_sc_mesh(),
        scratch_types=[
            pltpu.VMEM((nch, CH), jnp.int32),
            pltpu.VMEM((nch, CH), jnp.int32),
            pltpu.VMEM((CH, 128), jnp.float32),
            pltpu.VMEM_SHARED((n_pad, 128), jnp.float32),
            pltpu.SemaphoreType.DMA,
        ],
    )


def _prep_body(parts_ref, x_ref, wt_ref, hs_ref, dis_ref):
    deg = parts_ref[0, :] + parts_ref[1, :] + 1.0
    dis = lax.rsqrt(deg)
    h = jnp.dot(x_ref[...], wt_ref[...], preferred_element_type=jnp.float32)
    hs_ref[...] = h * dis[:, None]
    dis_ref[...] = dis


def _mid_body(parts_ref, hs_ref, dis_ref, wt_ref, out_ref):
    dis = dis_ref[...]
    agg = parts_ref[0] + parts_ref[1] + hs_ref[...]
    a = jnp.maximum(agg * dis[:, None], 0.0)
    h = jnp.dot(a, wt_ref[...], preferred_element_type=jnp.float32)
    out_ref[...] = h * dis[:, None]


def _final_body(parts_ref, hs_ref, dis_ref, wt_ref, b_ref, out_ref):
    dis = dis_ref[...]
    agg = parts_ref[0] + parts_ref[1] + hs_ref[...]
    a = agg * dis[:, None]
    out_ref[...] = (
        jnp.dot(a, wt_ref[...], preferred_element_type=jnp.float32)
        + b_ref[...]
    )


def kernel(x, edge_index, W1, W2, Wh, bh):
    n, d = x.shape
    e = edge_index.shape[1]
    blk = 512
    n_pad = ((n + blk - 1) // blk) * blk
    if n_pad == n:
        n_pad += blk
    nch = -(-e // (NW * CH))
    e_pad = NW * CH * nch
    sr = n_pad // NS

    row = jnp.concatenate(
        [edge_index[0], jnp.full((e_pad - e,), n, jnp.int32)]
    ).reshape(NW * nch, CH)
    col = jnp.concatenate(
        [edge_index[1], jnp.full((e_pad - e,), n, jnp.int32)]
    ).reshape(NW * nch, CH)

    ones_hbm = jnp.ones((CH,), jnp.float32)
    zeros1 = jnp.zeros((sr,), jnp.float32)
    zeros2 = jnp.zeros((sr, 128), jnp.float32)

    deg_parts = _deg_kernel(n_pad, nch)(col, ones_hbm, zeros1)
    agg = _agg_kernel(n_pad, nch)

    xpad = jnp.pad(x, ((0, n_pad - n), (0, 0)))

    grid = (n_pad // blk,)
    hs1, dis = pl.pallas_call(
        _prep_body,
        grid=grid,
        in_specs=[
            pl.BlockSpec((NC, blk), lambda i: (0, i)),
            pl.BlockSpec((blk, d), lambda i: (i, 0)),
            pl.BlockSpec((d, d), lambda i: (0, 0)),
        ],
        out_specs=[
            pl.BlockSpec((blk, d), lambda i: (i, 0)),
            pl.BlockSpec((blk,), lambda i: (i,)),
        ],
        out_shape=[
            jax.ShapeDtypeStruct((n_pad, d), jnp.float32),
            jax.ShapeDtypeStruct((n_pad,), jnp.float32),
        ],
    )(deg_parts, xpad, W1.T)

    parts1 = agg(hs1, row, col, zeros2)

    hs2 = pl.pallas_call(
        _mid_body,
        grid=grid,
        in_specs=[
            pl.BlockSpec((NC, blk, d), lambda i: (0, i, 0)),
            pl.BlockSpec((blk, d), lambda i: (i, 0)),
            pl.BlockSpec((blk,), lambda i: (i,)),
            pl.BlockSpec((d, d), lambda i: (0, 0)),
        ],
        out_specs=pl.BlockSpec((blk, d), lambda i: (i, 0)),
        out_shape=jax.ShapeDtypeStruct((n_pad, d), jnp.float32),
    )(parts1, hs1, dis, W2.T)

    parts2 = agg(hs2, row, col, zeros2)

    out = pl.pallas_call(
        _final_body,
        grid=grid,
        in_specs=[
            pl.BlockSpec((NC, blk, d), lambda i: (0, i, 0)),
            pl.BlockSpec((blk, d), lambda i: (i, 0)),
            pl.BlockSpec((blk,), lambda i: (i,)),
            pl.BlockSpec((d, d), lambda i: (0, 0)),
            pl.BlockSpec((1, d), lambda i: (0, 0)),
        ],
        out_specs=pl.BlockSpec((blk, d), lambda i: (i, 0)),
        out_shape=jax.ShapeDtypeStruct((n_pad, d), jnp.float32),
    )(parts2, hs2, dis, Wh.T, bh[None, :])

    return out[:n]


# speedup vs baseline: 8.5055x; 8.5055x over previous
import jax
import jax.numpy as jnp
from jax import lax
from jax.experimental import pallas as pl
from jax.experimental.pallas import tpu as pltpu
from jax.experimental.pallas import tpu_sc as plsc

NC = 2
NS = 16
NW = NC * NS
CH = 128


def _sc_mesh():
    return plsc.VectorSubcoreMesh(
        core_axis_name="c", subcore_axis_name="s", num_cores=NC, num_subcores=NS
    )


def _deg_kernel(n_pad, nch):
    sr = n_pad // NS

    def body(col_hbm, ones_hbm, zeros_hbm, out_hbm, cidx_v, ones_v, deg_sh, sem):
        c = lax.axis_index("c")
        s = lax.axis_index("s")
        wid = s * NC + c
        pltpu.sync_copy(zeros_hbm, deg_sh.at[pl.ds(s * sr, sr)])
        pltpu.sync_copy(ones_hbm, ones_v)
        pltpu.sync_copy(col_hbm.at[pl.ds(wid * nch, nch)], cidx_v)
        plsc.subcore_barrier()

        def chunk(j, carry):
            pltpu.sync_copy(ones_v, deg_sh.at[cidx_v.at[j]], add=True)
            return carry

        lax.fori_loop(0, nch, chunk, 0)
        plsc.subcore_barrier()
        pltpu.sync_copy(deg_sh.at[pl.ds(s * sr, sr)],
                        out_hbm.at[c, pl.ds(s * sr, sr)])

    return pl.kernel(
        body,
        out_type=jax.ShapeDtypeStruct((NC, n_pad), jnp.float32),
        mesh=_sc_mesh(),
        scratch_types=[
            pltpu.VMEM((nch, CH), jnp.int32),
            pltpu.VMEM((CH,), jnp.float32),
            pltpu.VMEM_SHARED((n_pad,), jnp.float32),
            pltpu.SemaphoreType.DMA,
        ],
    )


def _agg_kernel(n_pad, nch):
    sr = n_pad // NS

    def body(hs_hbm, row_hbm, col_hbm, zeros_hbm, out_hbm,
             ridx_v, cidx_v, rows_v, agg_sh, sem):
        c = lax.axis_index("c")
        s = lax.axis_index("s")
        wid = s * NC + c
        pltpu.sync_copy(zeros_hbm, agg_sh.at[pl.ds(s * sr, sr)])
        pltpu.sync_copy(row_hbm.at[pl.ds(wid * nch, nch)], ridx_v)
        pltpu.sync_copy(col_hbm.at[pl.ds(wid * nch, nch)], cidx_v)
        plsc.subcore_barrier()

        def chunk(j, carry):
            pltpu.async_copy(hs_hbm.at[ridx_v.at[j]], rows_v, sem).wait()
            pltpu.sync_copy(rows_v, agg_sh.at[cidx_v.at[j]], add=True)
            return carry

        lax.fori_loop(0, nch, chunk, 0)
        plsc.subcore_barrier()
        pltpu.sync_copy(agg_sh.at[pl.ds(s * sr, sr)],
                        out_hbm.at[c, pl.ds(s * sr, sr), :])

    return pl.kernel(
        body,
        out_type=jax.ShapeDtypeStruct((NC, n_pad, 128), jnp.float32),
        mesh=_sc_mesh(),
        scratch_types=[
            pltpu.VMEM((nch, CH), jnp.int32),
            pltpu.VMEM((nch, CH), jnp.int32),
            pltpu.VMEM((CH, 128), jnp.float32),
            pltpu.VMEM_SHARED((n_pad, 128), jnp.float32),
            pltpu.SemaphoreType.DMA,
        ],
    )


def _prep_body(parts_ref, x_ref, wt_ref, hs_ref, dis_ref):
    deg = parts_ref[0, :] + parts_ref[1, :] + 1.0
    dis = lax.rsqrt(deg)
    h = jnp.dot(x_ref[...], wt_ref[...], preferred_element_type=jnp.float32)
    hs_ref[...] = h * dis[:, None]
    dis_ref[...] = dis


def _mid_body(parts_ref, hs_ref, dis_ref, wt_ref, out_ref):
    dis = dis_ref[...]
    agg = parts_ref[0] + parts_ref[1] + hs_ref[...]
    a = jnp.maximum(agg * dis[:, None], 0.0)
    h = jnp.dot(a, wt_ref[...], preferred_element_type=jnp.float32)
    out_ref[...] = h * dis[:, None]


def _final_body(parts_ref, hs_ref, dis_ref, wt_ref, b_ref, out_ref):
    dis = dis_ref[...]
    agg = parts_ref[0] + parts_ref[1] + hs_ref[...]
    a = agg * dis[:, None]
    out_ref[...] = (
        jnp.dot(a, wt_ref[...], preferred_element_type=jnp.float32)
        + b_ref[...]
    )


def kernel(x, edge_index, W1, W2, Wh, bh):
    n, d = x.shape
    e = edge_index.shape[1]
    blk = 512
    n_pad = ((n + blk - 1) // blk) * blk
    if n_pad == n:
        n_pad += blk
    nch = -(-e // (NW * CH))
    nch = ((nch + 7) // 8) * 8
    e_pad = NW * CH * nch
    sr = n_pad // NS

    row = jnp.concatenate(
        [edge_index[0], jnp.full((e_pad - e,), n, jnp.int32)]
    ).reshape(NW * nch, CH)
    col = jnp.concatenate(
        [edge_index[1], jnp.full((e_pad - e,), n, jnp.int32)]
    ).reshape(NW * nch, CH)

    ones_hbm = jnp.ones((CH,), jnp.float32)
    zeros1 = jnp.zeros((sr,), jnp.float32)
    zeros2 = jnp.zeros((sr, 128), jnp.float32)

    deg_parts = _deg_kernel(n_pad, nch)(col, ones_hbm, zeros1)
    agg = _agg_kernel(n_pad, nch)

    xpad = jnp.pad(x, ((0, n_pad - n), (0, 0)))

    grid = (n_pad // blk,)
    hs1, dis = pl.pallas_call(
        _prep_body,
        grid=grid,
        in_specs=[
            pl.BlockSpec((NC, blk), lambda i: (0, i)),
            pl.BlockSpec((blk, d), lambda i: (i, 0)),
            pl.BlockSpec((d, d), lambda i: (0, 0)),
        ],
        out_specs=[
            pl.BlockSpec((blk, d), lambda i: (i, 0)),
            pl.BlockSpec((blk,), lambda i: (i,)),
        ],
        out_shape=[
            jax.ShapeDtypeStruct((n_pad, d), jnp.float32),
            jax.ShapeDtypeStruct((n_pad,), jnp.float32),
        ],
    )(deg_parts, xpad, W1.T)

    parts1 = agg(hs1, row, col, zeros2)

    hs2 = pl.pallas_call(
        _mid_body,
        grid=grid,
        in_specs=[
            pl.BlockSpec((NC, blk, d), lambda i: (0, i, 0)),
            pl.BlockSpec((blk, d), lambda i: (i, 0)),
            pl.BlockSpec((blk,), lambda i: (i,)),
            pl.BlockSpec((d, d), lambda i: (0, 0)),
        ],
        out_specs=pl.BlockSpec((blk, d), lambda i: (i, 0)),
        out_shape=jax.ShapeDtypeStruct((n_pad, d), jnp.float32),
    )(parts1, hs1, dis, W2.T)

    parts2 = agg(hs2, row, col, zeros2)

    out = pl.pallas_call(
        _final_body,
        grid=grid,
        in_specs=[
            pl.BlockSpec((NC, blk, d), lambda i: (0, i, 0)),
            pl.BlockSpec((blk, d), lambda i: (i, 0)),
            pl.BlockSpec((blk,), lambda i: (i,)),
            pl.BlockSpec((d, d), lambda i: (0, 0)),
            pl.BlockSpec((1, d), lambda i: (0, 0)),
        ],
        out_specs=pl.BlockSpec((blk, d), lambda i: (i, 0)),
        out_shape=jax.ShapeDtypeStruct((n_pad, d), jnp.float32),
    )(parts2, hs2, dis, Wh.T, bh[None, :])

    return out[:n]
